# baseline (speedup 1.0000x reference)
import functools

import jax
import jax.numpy as jnp
from jax import lax
from jax.experimental import pallas as pl
from jax.experimental.pallas import tpu as pltpu
from jax.experimental.pallas import tpu_sc as plsc

D = 1024
L = 16
NCH = D // L
EPS = 1e-05
SEQ = 4096
CHUNK = 16


def _rsqrt(v):
    i = plsc.bitcast(v, jnp.int32)
    y = plsc.bitcast(jnp.int32(0x5F3759DF) - (i >> 1), jnp.float32)
    for _ in range(3):
        y = y * (1.5 - 0.5 * v * y * y)
    return y


_GATHER_DNUMS = lax.GatherDimensionNumbers(
    offset_dims=(), collapsed_slice_dims=(0,), start_index_map=(0,))


def _lane_shuffle(v, idx):
    return lax.gather(v, idx[:, None], _GATHER_DNUMS, slice_sizes=(1,),
                      mode=lax.GatherScatterMode.PROMISE_IN_BOUNDS)


def _lane_sum(v):
    lanes = lax.iota(jnp.int32, L)
    for sh in (8, 4, 2, 1):
        v = v + _lane_shuffle(v, lanes ^ sh)
    return v


@functools.lru_cache(maxsize=None)
def _make_sc_kernel(n_rows, c_rows):
    info = plsc.get_sparse_core_info()
    nw = info.num_cores * info.num_subcores
    per_w = n_rows // nw
    n_g = per_w // c_rows
    mesh = plsc.VectorSubcoreMesh(core_axis_name="c", subcore_axis_name="s")

    @functools.partial(
        pl.kernel,
        mesh=mesh,
        out_type=jax.ShapeDtypeStruct((n_rows, D), jnp.float32),
        compiler_params=pltpu.CompilerParams(needs_layout_passes=False),
        scratch_types=[
            pltpu.VMEM((per_w,), jnp.int32),
            pltpu.VMEM((c_rows, D), jnp.float32),
            pltpu.VMEM((c_rows, D), jnp.float32),
            pltpu.VMEM((D,), jnp.float32),
            pltpu.VMEM((D,), jnp.float32),
            pltpu.VMEM((D,), jnp.float32),
            pltpu.SemaphoreType.DMA,
        ],
    )
    def k(ids_hbm, word_hbm, pos_hbm, type_hbm, w_hbm, b_hbm, out_hbm,
          idx_v, word_v, pos_v, type_v, w_v, b_v, sem):
        wid = lax.axis_index("s") * info.num_cores + lax.axis_index("c")
        base = wid * per_w
        s0 = lax.rem(base, SEQ)
        pltpu.sync_copy(ids_hbm.at[pl.ds(base, per_w)], idx_v)
        pltpu.sync_copy(type_hbm, type_v)
        pltpu.sync_copy(w_hbm, w_v)
        pltpu.sync_copy(b_hbm, b_v)

        def g_body(g, carry):
            row0 = g * c_rows
            gat = pltpu.async_copy(
                word_hbm.at[idx_v.at[pl.ds(row0, c_rows)]], word_v, sem)
            pltpu.sync_copy(pos_hbm.at[pl.ds(s0 + row0, c_rows)], pos_v)
            gat.wait()

            def r_body(r, rc):
                vs = jnp.zeros((L,), jnp.float32)
                vq = jnp.zeros((L,), jnp.float32)
                for j in range(NCH):
                    x = (word_v[r, pl.ds(j * L, L)]
                         + pos_v[r, pl.ds(j * L, L)]
                         + type_v[pl.ds(j * L, L)])
                    vs = vs + x
                    vq = vq + x * x
                mean_v = _lane_sum(vs) * (1.0 / D)
                var_v = _lane_sum(vq) * (1.0 / D) - mean_v * mean_v
                rstd_v = _rsqrt(var_v + EPS)
                for j in range(NCH):
                    x = (word_v[r, pl.ds(j * L, L)]
                         + pos_v[r, pl.ds(j * L, L)]
                         + type_v[pl.ds(j * L, L)])
                    y = ((x - mean_v) * rstd_v * w_v[pl.ds(j * L, L)]
                         + b_v[pl.ds(j * L, L)])
                    word_v[r, pl.ds(j * L, L)] = y
                return rc

            lax.fori_loop(0, c_rows, r_body, 0)
            pltpu.sync_copy(word_v, out_hbm.at[pl.ds(base + row0, c_rows)])
            return carry

        lax.fori_loop(0, n_g, g_body, 0)

    return k


def kernel(input_ids, word_table, pos_table, type_table, ln_weight, ln_bias):
    b, s = input_ids.shape
    ids_flat = jnp.reshape(input_ids.astype(jnp.int32), (b * s,))
    type_row = jnp.reshape(type_table, (D,))
    k = _make_sc_kernel(b * s, CHUNK)
    out = k(ids_flat, word_table, pos_table, type_row, ln_weight, ln_bias)
    return jnp.reshape(out, (b, s, D))
